# baseline (speedup 1.0000x reference)
import functools

import jax
import jax.numpy as jnp
from jax import lax
from jax.experimental import pallas as pl
from jax.experimental.pallas import tpu as pltpu
from jax.experimental.pallas import tpu_sc as plsc

Q = 1024
K = 100000
D = 128
TOPK = 32
KBLK = 2048
KPAD = 100352
NBLK = KPAD // KBLK
C = 128
NCHUNK = KPAD // C
CBLK = KBLK // C
NEG = -3.0e38

NW = 32
QPW = Q // NW
CAND = TOPK * C
NSEG = CAND // 256


def _sim_body(q_ref, k_ref, sim_ref, cmax_ref, qn_ref):
    i = pl.program_id(0)

    @pl.when(i == 0)
    def _():
        qb = q_ref[...]
        qn_ref[...] = qb * jax.lax.rsqrt(
            jnp.sum(qb * qb, axis=-1, keepdims=True))

    kb = k_ref[...]
    kn = kb * jax.lax.rsqrt(jnp.sum(kb * kb, axis=-1, keepdims=True))
    s = jnp.dot(qn_ref[...], kn.T, preferred_element_type=jnp.float32)
    gcol = i * KBLK + jax.lax.broadcasted_iota(jnp.int32, (Q, KBLK), 1)
    s = jnp.where(gcol < K, s, NEG)
    sim_ref[...] = s
    cmax_ref[...] = jnp.max(s.reshape(Q, CBLK, C), axis=-1)[None]


def _allmax(x, tmp, iota):
    for s in (8, 4, 2, 1):
        tmp[pl.ds(0, 16)] = x
        x = jnp.maximum(x, plsc.load_gather(tmp, [iota ^ s]))
    return x[0]


def _ffs(mask):
    return plsc.all_reduce_ffs(mask)[0]


def _sc_topk_body(table_hbm, fidx_hbm, tv_hbm, ti_hbm,
                  fid_v, idx_q, xbuf, m1, tmp, tvall, tiall, sem):
    wid = lax.axis_index("s") * 2 + lax.axis_index("c")
    pltpu.sync_copy(fidx_hbm.at[pl.ds(wid * QPW, QPW)], fid_v)
    iota = lax.iota(jnp.int32, 16)
    negv = jnp.full((16,), NEG, jnp.float32)

    def per_query(q, carry):
        qglob = wid * QPW + q
        idx_q[pl.ds(0, 16)] = fid_v[q, pl.ds(0, 16)]
        idx_q[pl.ds(16, 16)] = fid_v[q, pl.ds(16, 16)]
        pltpu.async_copy(table_hbm.at[idx_q], xbuf, sem).wait()

        def seg_max(jj, c2):
            macc = negv
            for v in range(16):
                macc = jnp.maximum(
                    macc, xbuf[jj * 2 + v // 8, pl.ds((v % 8) * 16, 16)])
            m1[pl.ds(jj * 16, 16)] = macc
            return c2

        lax.fori_loop(0, NSEG, seg_max, 0)

        def rnd(r, c2):
            m2 = m1[pl.ds(0, 16)]
            for jj in range(1, NSEG):
                m2 = jnp.maximum(m2, m1[pl.ds(jj * 16, 16)])
            gmax = _allmax(m2, tmp, iota)
            lstar = _ffs(m2 == gmax)
            m1g = plsc.load_gather(m1, [iota * 16 + lstar])
            jstar = _ffs(m1g == gmax)
            pvec = (jstar * 16 + iota) * 16 + lstar
            rowv = pvec // C
            colv = pvec % C
            segv = plsc.load_gather(xbuf, [rowv, colv])
            vstar = _ffs(segv == gmax)
            p = (jstar * 16 + vstar) * 16 + lstar
            row = p // C
            col = p % C
            fid = plsc.load_gather(
                fid_v, [jnp.broadcast_to(q, (16,)).astype(jnp.int32),
                        jnp.broadcast_to(row, (16,))])[0]
            chunk = fid - qglob * NCHUNK
            gi = chunk * C + col
            rhi = (r // 16) * 16
            rlo = r % 16
            tvall[q, pl.ds(rhi, 16)] = jnp.where(
                iota == rlo, gmax, tvall[q, pl.ds(rhi, 16)])
            tiall[q, pl.ds(rhi, 16)] = jnp.where(
                iota == rlo, gi, tiall[q, pl.ds(rhi, 16)])
            plsc.store_scatter(xbuf, [rowv, colv], negv, mask=iota == vstar)
            nsm = _allmax(jnp.where(iota == vstar, NEG, segv), tmp, iota)
            m1[pl.ds(jstar * 16, 16)] = jnp.where(
                iota == lstar, nsm, m1[pl.ds(jstar * 16, 16)])
            return c2

        lax.fori_loop(0, TOPK, rnd, 0)
        return carry

    lax.fori_loop(0, QPW, per_query, 0)
    pltpu.sync_copy(tvall, tv_hbm.at[pl.ds(wid * QPW, QPW)])
    pltpu.sync_copy(tiall, ti_hbm.at[pl.ds(wid * QPW, QPW)])


def _sc_topk(table, fidx):
    mesh = plsc.VectorSubcoreMesh(core_axis_name="c", subcore_axis_name="s")
    gk = functools.partial(
        pl.kernel, mesh=mesh,
        out_type=[
            jax.ShapeDtypeStruct((Q, TOPK), jnp.float32),
            jax.ShapeDtypeStruct((Q, TOPK), jnp.int32),
        ],
        scratch_types=[
            pltpu.VMEM((QPW, TOPK), jnp.int32),
            pltpu.VMEM((TOPK,), jnp.int32),
            pltpu.VMEM((TOPK, C), jnp.float32),
            pltpu.VMEM((NSEG * 16,), jnp.float32),
            pltpu.VMEM((16,), jnp.float32),
            pltpu.VMEM((QPW, TOPK), jnp.float32),
            pltpu.VMEM((QPW, TOPK), jnp.int32),
            pltpu.SemaphoreType.DMA,
        ],
        compiler_params=pltpu.CompilerParams(needs_layout_passes=False),
    )(_sc_topk_body)
    return gk(table, fidx)


QB = 256


def _chunk_topk_body(cmax_ref, fidx_ref):
    x = cmax_ref[...]
    cidx = jax.lax.broadcasted_iota(jnp.int32, (QB, NCHUNK), 1)
    qidx = (pl.program_id(0) * QB
            + jax.lax.broadcasted_iota(jnp.int32, (QB, 1), 0))
    picks = []
    for _ in range(TOPK):
        m = jnp.max(x, axis=-1, keepdims=True)
        sel = jnp.where(x == m, cidx, jnp.int32(2**30))
        am = jnp.min(sel, axis=-1, keepdims=True)
        picks.append(am)
        x = jnp.where(cidx == am, NEG, x)
    fidx_ref[...] = jnp.concatenate(picks, axis=-1) + qidx * NCHUNK


def kernel(queries, keys):
    pad = KPAD - K
    keys_p = jnp.concatenate(
        [keys, jnp.ones((pad, D), jnp.float32)], axis=0)

    sim, cmax = pl.pallas_call(
        _sim_body,
        grid=(NBLK,),
        in_specs=[
            pl.BlockSpec((Q, D), lambda i: (0, 0)),
            pl.BlockSpec((KBLK, D), lambda i: (i, 0)),
        ],
        out_specs=[
            pl.BlockSpec((Q, KBLK), lambda i: (0, i)),
            pl.BlockSpec((1, Q, CBLK), lambda i: (i, 0, 0)),
        ],
        out_shape=[
            jax.ShapeDtypeStruct((Q, KPAD), jnp.float32),
            jax.ShapeDtypeStruct((NBLK, Q, CBLK), jnp.float32),
        ],
        scratch_shapes=[pltpu.VMEM((Q, D), jnp.float32)],
    )(queries, keys_p)
    cmax = cmax.transpose(1, 0, 2).reshape(Q, NCHUNK)
    fidx = pl.pallas_call(
        _chunk_topk_body,
        grid=(Q // QB,),
        in_specs=[pl.BlockSpec((QB, NCHUNK), lambda i: (i, 0))],
        out_specs=pl.BlockSpec((QB, TOPK), lambda i: (i, 0)),
        out_shape=jax.ShapeDtypeStruct((Q, TOPK), jnp.int32),
    )(cmax)

    tv, ti = _sc_topk(sim.reshape(Q * NCHUNK, C), fidx)
    return tv, ti


# speedup vs baseline: 1.0321x; 1.0321x over previous
import functools

import jax
import jax.numpy as jnp
from jax import lax
from jax.experimental import pallas as pl
from jax.experimental.pallas import tpu as pltpu
from jax.experimental.pallas import tpu_sc as plsc

Q = 1024
K = 100000
D = 128
TOPK = 32
KBLK = 2048
KPAD = 100352
NBLK = KPAD // KBLK
C = 128
NCHUNK = KPAD // C
CBLK = KBLK // C
NEG = -3.0e38

NW = 32
QPW = Q // NW
CAND = TOPK * C
NSEG = CAND // 256


def _sim_body(q_ref, k_ref, sim_ref, cmax_ref, qn_ref):
    i = pl.program_id(0)

    @pl.when(i == 0)
    def _():
        qb = q_ref[...]
        qn_ref[...] = qb * jax.lax.rsqrt(
            jnp.sum(qb * qb, axis=-1, keepdims=True))

    kb = k_ref[...]
    kn = kb * jax.lax.rsqrt(jnp.sum(kb * kb, axis=-1, keepdims=True))
    s = jnp.dot(qn_ref[...], kn.T, preferred_element_type=jnp.float32)
    gcol = i * KBLK + jax.lax.broadcasted_iota(jnp.int32, (Q, KBLK), 1)
    s = jnp.where(gcol < K, s, NEG)
    sim_ref[...] = s
    cmax_ref[...] = jnp.max(s.reshape(Q, CBLK, C), axis=-1)[None]


def _allmax(x, tmp, iota):
    for s in (8, 4, 2, 1):
        tmp[pl.ds(0, 16)] = x
        x = jnp.maximum(x, plsc.load_gather(tmp, [iota ^ s]))
    return x[0]


def _ffs(mask):
    return plsc.all_reduce_ffs(mask)[0]


def _sc_topk_body(table_hbm, fidx_hbm, tv_hbm, ti_hbm,
                  fid_v, idx_q, xbuf, m1, tmp, tvall, tiall, sem):
    wid = lax.axis_index("s") * 2 + lax.axis_index("c")
    pltpu.sync_copy(fidx_hbm.at[pl.ds(wid * QPW, QPW)], fid_v)
    iota = lax.iota(jnp.int32, 16)
    negv = jnp.full((16,), NEG, jnp.float32)

    idx_q[0, pl.ds(0, 16)] = fid_v[0, pl.ds(0, 16)]
    idx_q[0, pl.ds(16, 16)] = fid_v[0, pl.ds(16, 16)]
    pltpu.async_copy(table_hbm.at[idx_q.at[0]], xbuf.at[0], sem)

    def per_query(q, carry):
        qglob = wid * QPW + q
        b = q % 2
        nb = 1 - b
        pltpu.make_async_copy(
            table_hbm.at[idx_q.at[b]], xbuf.at[b], sem).wait()

        @pl.when(q < QPW - 1)
        def _():
            idx_q[nb, pl.ds(0, 16)] = fid_v[q + 1, pl.ds(0, 16)]
            idx_q[nb, pl.ds(16, 16)] = fid_v[q + 1, pl.ds(16, 16)]
            pltpu.async_copy(table_hbm.at[idx_q.at[nb]], xbuf.at[nb], sem)

        bs = jnp.broadcast_to(b, (16,)).astype(jnp.int32)

        def seg_max(jj, c2):
            macc = negv
            for v in range(16):
                macc = jnp.maximum(
                    macc, xbuf[b, jj * 2 + v // 8, pl.ds((v % 8) * 16, 16)])
            m1[pl.ds(jj * 16, 16)] = macc
            return c2

        lax.fori_loop(0, NSEG, seg_max, 0)

        def rnd(r, c2):
            m2 = m1[pl.ds(0, 16)]
            for jj in range(1, NSEG):
                m2 = jnp.maximum(m2, m1[pl.ds(jj * 16, 16)])
            gmax = _allmax(m2, tmp, iota)
            lstar = _ffs(m2 == gmax)
            m1g = plsc.load_gather(m1, [iota * 16 + lstar])
            jstar = _ffs(m1g == gmax)
            pvec = (jstar * 16 + iota) * 16 + lstar
            rowv = pvec // C
            colv = pvec % C
            segv = plsc.load_gather(xbuf, [bs, rowv, colv])
            vstar = _ffs(segv == gmax)
            p = (jstar * 16 + vstar) * 16 + lstar
            row = p // C
            col = p % C
            fid = plsc.load_gather(
                fid_v, [jnp.broadcast_to(q, (16,)).astype(jnp.int32),
                        jnp.broadcast_to(row, (16,))])[0]
            chunk = fid - qglob * NCHUNK
            gi = chunk * C + col
            rhi = (r // 16) * 16
            rlo = r % 16
            tvall[q, pl.ds(rhi, 16)] = jnp.where(
                iota == rlo, gmax, tvall[q, pl.ds(rhi, 16)])
            tiall[q, pl.ds(rhi, 16)] = jnp.where(
                iota == rlo, gi, tiall[q, pl.ds(rhi, 16)])
            plsc.store_scatter(xbuf, [bs, rowv, colv], negv, mask=iota == vstar)
            nsm = _allmax(jnp.where(iota == vstar, NEG, segv), tmp, iota)
            m1[pl.ds(jstar * 16, 16)] = jnp.where(
                iota == lstar, nsm, m1[pl.ds(jstar * 16, 16)])
            return c2

        lax.fori_loop(0, TOPK, rnd, 0)
        return carry

    lax.fori_loop(0, QPW, per_query, 0)
    pltpu.sync_copy(tvall, tv_hbm.at[pl.ds(wid * QPW, QPW)])
    pltpu.sync_copy(tiall, ti_hbm.at[pl.ds(wid * QPW, QPW)])


def _sc_topk(table, fidx):
    mesh = plsc.VectorSubcoreMesh(core_axis_name="c", subcore_axis_name="s")
    gk = functools.partial(
        pl.kernel, mesh=mesh,
        out_type=[
            jax.ShapeDtypeStruct((Q, TOPK), jnp.float32),
            jax.ShapeDtypeStruct((Q, TOPK), jnp.int32),
        ],
        scratch_types=[
            pltpu.VMEM((QPW, TOPK), jnp.int32),
            pltpu.VMEM((2, TOPK), jnp.int32),
            pltpu.VMEM((2, TOPK, C), jnp.float32),
            pltpu.VMEM((NSEG * 16,), jnp.float32),
            pltpu.VMEM((16,), jnp.float32),
            pltpu.VMEM((QPW, TOPK), jnp.float32),
            pltpu.VMEM((QPW, TOPK), jnp.int32),
            pltpu.SemaphoreType.DMA,
        ],
        compiler_params=pltpu.CompilerParams(needs_layout_passes=False),
    )(_sc_topk_body)
    return gk(table, fidx)


QB = 256


def _chunk_topk_body(cmax_ref, fidx_ref):
    x = cmax_ref[...]
    cidx = jax.lax.broadcasted_iota(jnp.int32, (QB, NCHUNK), 1)
    qidx = (pl.program_id(0) * QB
            + jax.lax.broadcasted_iota(jnp.int32, (QB, 1), 0))
    picks = []
    for _ in range(TOPK):
        m = jnp.max(x, axis=-1, keepdims=True)
        sel = jnp.where(x == m, cidx, jnp.int32(2**30))
        am = jnp.min(sel, axis=-1, keepdims=True)
        picks.append(am)
        x = jnp.where(cidx == am, NEG, x)
    fidx_ref[...] = jnp.concatenate(picks, axis=-1) + qidx * NCHUNK


def kernel(queries, keys):
    pad = KPAD - K
    keys_p = jnp.concatenate(
        [keys, jnp.ones((pad, D), jnp.float32)], axis=0)

    sim, cmax = pl.pallas_call(
        _sim_body,
        grid=(NBLK,),
        in_specs=[
            pl.BlockSpec((Q, D), lambda i: (0, 0)),
            pl.BlockSpec((KBLK, D), lambda i: (i, 0)),
        ],
        out_specs=[
            pl.BlockSpec((Q, KBLK), lambda i: (0, i)),
            pl.BlockSpec((1, Q, CBLK), lambda i: (i, 0, 0)),
        ],
        out_shape=[
            jax.ShapeDtypeStruct((Q, KPAD), jnp.float32),
            jax.ShapeDtypeStruct((NBLK, Q, CBLK), jnp.float32),
        ],
        scratch_shapes=[pltpu.VMEM((Q, D), jnp.float32)],
    )(queries, keys_p)
    cmax = cmax.transpose(1, 0, 2).reshape(Q, NCHUNK)
    fidx = pl.pallas_call(
        _chunk_topk_body,
        grid=(Q // QB,),
        in_specs=[pl.BlockSpec((QB, NCHUNK), lambda i: (i, 0))],
        out_specs=pl.BlockSpec((QB, TOPK), lambda i: (i, 0)),
        out_shape=jax.ShapeDtypeStruct((Q, TOPK), jnp.int32),
    )(cmax)

    tv, ti = _sc_topk(sim.reshape(Q * NCHUNK, C), fidx)
    return tv, ti


# speedup vs baseline: 1.1027x; 1.0684x over previous
import functools

import jax
import jax.numpy as jnp
from jax import lax
from jax.experimental import pallas as pl
from jax.experimental.pallas import tpu as pltpu
from jax.experimental.pallas import tpu_sc as plsc

Q = 1024
K = 100000
D = 128
TOPK = 32
KBLK = 2048
KPAD = 100352
NBLK = KPAD // KBLK
C = 128
NCHUNK = KPAD // C
CBLK = KBLK // C
NEG = -3.0e38

NW = 32
QPW = Q // NW
CAND = TOPK * C
NSEG = CAND // 256


def _sim_body(q_ref, k_ref, sim_ref, cmax_ref, qn_ref):
    i = pl.program_id(0)

    @pl.when(i == 0)
    def _():
        qb = q_ref[...]
        qn_ref[...] = qb * jax.lax.rsqrt(
            jnp.sum(qb * qb, axis=-1, keepdims=True))

    kb = k_ref[...]
    kn = kb * jax.lax.rsqrt(jnp.sum(kb * kb, axis=-1, keepdims=True))
    s = jnp.dot(qn_ref[...], kn.T, preferred_element_type=jnp.float32)
    gcol = i * KBLK + jax.lax.broadcasted_iota(jnp.int32, (Q, KBLK), 1)
    s = jnp.where(gcol < K, s, NEG)
    sim_ref[...] = s
    cmax_ref[...] = jnp.max(s.reshape(Q, CBLK, C), axis=-1)[None]


def _vmax(x):
    return jnp.sort(x)[15]


def _ffs(mask):
    return plsc.all_reduce_ffs(mask)[0]


def _sc_topk_body(table_hbm, fidx_hbm, tv_hbm, ti_hbm,
                  fid_v, idx_q, xbuf, m1, tvall, tiall, sem):
    wid = lax.axis_index("s") * 2 + lax.axis_index("c")
    pltpu.sync_copy(fidx_hbm.at[pl.ds(wid * QPW, QPW)], fid_v)
    iota = lax.iota(jnp.int32, 16)
    negv = jnp.full((16,), NEG, jnp.float32)

    idx_q[0, pl.ds(0, 16)] = fid_v[0, pl.ds(0, 16)]
    idx_q[0, pl.ds(16, 16)] = fid_v[0, pl.ds(16, 16)]
    pltpu.async_copy(table_hbm.at[idx_q.at[0]], xbuf.at[0], sem)

    def per_query(q, carry):
        qglob = wid * QPW + q
        b = q % 2
        nb = 1 - b
        pltpu.make_async_copy(
            table_hbm.at[idx_q.at[b]], xbuf.at[b], sem).wait()

        @pl.when(q < QPW - 1)
        def _():
            idx_q[nb, pl.ds(0, 16)] = fid_v[q + 1, pl.ds(0, 16)]
            idx_q[nb, pl.ds(16, 16)] = fid_v[q + 1, pl.ds(16, 16)]
            pltpu.async_copy(table_hbm.at[idx_q.at[nb]], xbuf.at[nb], sem)

        bs = jnp.broadcast_to(b, (16,)).astype(jnp.int32)

        def seg_max(jj, c2):
            macc = negv
            for v in range(16):
                macc = jnp.maximum(
                    macc, xbuf[b, jj * 2 + v // 8, pl.ds((v % 8) * 16, 16)])
            m1[pl.ds(jj * 16, 16)] = macc
            return c2

        lax.fori_loop(0, NSEG, seg_max, 0)

        m2_0 = m1[pl.ds(0, 16)]
        for jj in range(1, NSEG):
            m2_0 = jnp.maximum(m2_0, m1[pl.ds(jj * 16, 16)])

        def rnd(r, m2):
            gmax = _vmax(m2)
            lstar = _ffs(m2 == gmax)
            m1g = plsc.load_gather(m1, [iota * 16 + lstar])
            jstar = _ffs(m1g == gmax)
            pvec = (jstar * 16 + iota) * 16 + lstar
            rowv = pvec // C
            colv = pvec % C
            segv = plsc.load_gather(xbuf, [bs, rowv, colv])
            vstar = _ffs(segv == gmax)
            p = (jstar * 16 + vstar) * 16 + lstar
            row = p // C
            col = p % C
            fid = plsc.load_gather(
                fid_v, [jnp.broadcast_to(q, (16,)).astype(jnp.int32),
                        jnp.broadcast_to(row, (16,))])[0]
            chunk = fid - qglob * NCHUNK
            gi = chunk * C + col
            rhi = (r // 16) * 16
            rlo = r % 16
            tvall[q, pl.ds(rhi, 16)] = jnp.where(
                iota == rlo, gmax, tvall[q, pl.ds(rhi, 16)])
            tiall[q, pl.ds(rhi, 16)] = jnp.where(
                iota == rlo, gi, tiall[q, pl.ds(rhi, 16)])
            plsc.store_scatter(xbuf, [bs, rowv, colv], negv, mask=iota == vstar)
            nsm = _vmax(jnp.where(iota == vstar, NEG, segv))
            m1g_new = jnp.where(iota == jstar, nsm, m1g)
            m1[pl.ds(jstar * 16, 16)] = jnp.where(
                iota == lstar, nsm, m1[pl.ds(jstar * 16, 16)])
            return jnp.where(iota == lstar, _vmax(m1g_new), m2)

        lax.fori_loop(0, TOPK, rnd, m2_0)
        return carry

    lax.fori_loop(0, QPW, per_query, 0)
    pltpu.sync_copy(tvall, tv_hbm.at[pl.ds(wid * QPW, QPW)])
    pltpu.sync_copy(tiall, ti_hbm.at[pl.ds(wid * QPW, QPW)])


def _sc_topk(table, fidx):
    mesh = plsc.VectorSubcoreMesh(core_axis_name="c", subcore_axis_name="s")
    gk = functools.partial(
        pl.kernel, mesh=mesh,
        out_type=[
            jax.ShapeDtypeStruct((Q, TOPK), jnp.float32),
            jax.ShapeDtypeStruct((Q, TOPK), jnp.int32),
        ],
        scratch_types=[
            pltpu.VMEM((QPW, TOPK), jnp.int32),
            pltpu.VMEM((2, TOPK), jnp.int32),
            pltpu.VMEM((2, TOPK, C), jnp.float32),
            pltpu.VMEM((NSEG * 16,), jnp.float32),
            pltpu.VMEM((QPW, TOPK), jnp.float32),
            pltpu.VMEM((QPW, TOPK), jnp.int32),
            pltpu.SemaphoreType.DMA,
        ],
        compiler_params=pltpu.CompilerParams(needs_layout_passes=False),
    )(_sc_topk_body)
    return gk(table, fidx)


QB = 256


def _chunk_topk_body(cmax_ref, fidx_ref):
    x = cmax_ref[...]
    cidx = jax.lax.broadcasted_iota(jnp.int32, (QB, NCHUNK), 1)
    qidx = (pl.program_id(0) * QB
            + jax.lax.broadcasted_iota(jnp.int32, (QB, 1), 0))
    picks = []
    for _ in range(TOPK):
        m = jnp.max(x, axis=-1, keepdims=True)
        sel = jnp.where(x == m, cidx, jnp.int32(2**30))
        am = jnp.min(sel, axis=-1, keepdims=True)
        picks.append(am)
        x = jnp.where(cidx == am, NEG, x)
    fidx_ref[...] = jnp.concatenate(picks, axis=-1) + qidx * NCHUNK


def kernel(queries, keys):
    pad = KPAD - K
    keys_p = jnp.concatenate(
        [keys, jnp.ones((pad, D), jnp.float32)], axis=0)

    sim, cmax = pl.pallas_call(
        _sim_body,
        grid=(NBLK,),
        in_specs=[
            pl.BlockSpec((Q, D), lambda i: (0, 0)),
            pl.BlockSpec((KBLK, D), lambda i: (i, 0)),
        ],
        out_specs=[
            pl.BlockSpec((Q, KBLK), lambda i: (0, i)),
            pl.BlockSpec((1, Q, CBLK), lambda i: (i, 0, 0)),
        ],
        out_shape=[
            jax.ShapeDtypeStruct((Q, KPAD), jnp.float32),
            jax.ShapeDtypeStruct((NBLK, Q, CBLK), jnp.float32),
        ],
        scratch_shapes=[pltpu.VMEM((Q, D), jnp.float32)],
    )(queries, keys_p)
    cmax = cmax.transpose(1, 0, 2).reshape(Q, NCHUNK)
    fidx = pl.pallas_call(
        _chunk_topk_body,
        grid=(Q // QB,),
        in_specs=[pl.BlockSpec((QB, NCHUNK), lambda i: (i, 0))],
        out_specs=pl.BlockSpec((QB, TOPK), lambda i: (i, 0)),
        out_shape=jax.ShapeDtypeStruct((Q, TOPK), jnp.int32),
    )(cmax)

    tv, ti = _sc_topk(sim.reshape(Q * NCHUNK, C), fidx)
    return tv, ti
